# baseline (speedup 1.0000x reference)
import functools

import jax
import jax.numpy as jnp
from jax import lax
from jax.experimental import pallas as pl
from jax.experimental.pallas import tpu as pltpu
from jax.experimental.pallas import tpu_sc as plsc

VOCAB = 1_000_000
DIM = 64
BATCH = 16384

NUM_CORES = 2
NUM_SUBCORES = 16
NUM_WORKERS = NUM_CORES * NUM_SUBCORES
BPW = BATCH // NUM_WORKERS
CHUNK = 128
NCHUNK = BPW // CHUNK
LANES = 16

RSPAN = 31360
BSPAN = 2048
NBUCK = 16
TAIL = 999936
MAXSTART = 997888
MCAP = 2048
M2CAP = 2432
SUB = 128
NIDX = BATCH // 8

_mesh = plsc.VectorSubcoreMesh(core_axis_name="c", subcore_axis_name="s")


def _wid():
    return lax.axis_index("s") * NUM_CORES + lax.axis_index("c")


@functools.partial(
    pl.kernel,
    out_type=(jax.ShapeDtypeStruct((BATCH, 128), jnp.float32),
              jax.ShapeDtypeStruct((BATCH, 128), jnp.float32)),
    mesh=_mesh,
    compiler_params=pltpu.CompilerParams(needs_layout_passes=False,
                                         use_tc_tiling_on_sc=True),
    scratch_types=[
        pltpu.VMEM((NIDX,), jnp.int32),
        pltpu.VMEM((MCAP + 32,), jnp.int32),
        pltpu.VMEM((MCAP + 32,), jnp.int32),
        pltpu.VMEM((M2CAP,), jnp.int32),
        pltpu.VMEM((M2CAP,), jnp.int32),
        pltpu.VMEM((8, BSPAN), jnp.float32),
        pltpu.VMEM((8, BSPAN), jnp.float32),
        pltpu.VMEM((8, BSPAN), jnp.float32),
        pltpu.VMEM((8, BSPAN), jnp.float32),
        pltpu.VMEM((SUB, 128), jnp.float32),
        pltpu.VMEM((1, 128), jnp.int32),
        pltpu.VMEM((4096,), jnp.float32),
        pltpu.SMEM((64,), jnp.int32),
        pltpu.SemaphoreType.DMA,
        pltpu.SemaphoreType.DMA,
        pltpu.SemaphoreType.DMA,
        pltpu.SemaphoreType.DMA,
        pltpu.SemaphoreType.DMA,
    ],
)
def _sweep(i_hbm, j_hbm, wi_hbm, wj_hbm, witail_hbm, wjtail_hbm,
           owi_hbm, owj_hbm,
           idx_buf, match_r, match_p, m2r, m2p, chunk_buf, chunk_buf2,
           chunk_buf3, chunk_buf4, grows, pos2d, tail_v, smem, sem,
           sem_a, sem_b, sem_c, sem_d):
    wid = _wid()
    lo = wid * RSPAN
    hi = jnp.minimum(lo + RSPAN, VOCAB)
    lane = lax.iota(jnp.int32, LANES)

    for t in range(2):
        idx_hbm = (i_hbm, j_hbm)[t]
        tab_hbm = (wi_hbm, wj_hbm)[t]
        tl_hbm = (witail_hbm, wjtail_hbm)[t]
        out_hbm = (owi_hbm, owj_hbm)[t]

        pltpu.sync_copy(tl_hbm, tail_v)

        bufs = (chunk_buf, chunk_buf2, chunk_buf3, chunk_buf4)
        sems = (sem_a, sem_b, sem_c, sem_d)

        def _cstart_of(c):
            return pl.multiple_of(jnp.minimum(lo + c * BSPAN, MAXSTART), 128)

        cs0 = _cstart_of(jnp.int32(0))
        for p in range(4):
            pltpu.async_copy(tab_hbm.at[pl.ds(p * 8, 8), pl.ds(cs0, BSPAN)],
                             bufs[p], sems[p])

        def outer(c8, off):
            pltpu.sync_copy(idx_hbm.at[pl.ds(c8 * NIDX, NIDX)], idx_buf)

            def cb(v, off):
                x = idx_buf[pl.ds(v * LANES, LANES)]
                m = (x >= lo) & (x < hi)
                cnt = jnp.sum(m.astype(jnp.int32))
                offw = jnp.minimum(off, MCAP)
                plsc.store_compressed(match_r.at[pl.ds(offw, LANES)], x, mask=m)
                pos = c8 * NIDX + v * LANES + lane
                plsc.store_compressed(match_p.at[pl.ds(offw, LANES)], pos, mask=m)
                return off + cnt

            return lax.fori_loop(0, NIDX // LANES, cb, off)

        off = lax.fori_loop(0, 8, outer, 0)
        off = jnp.minimum(off, MCAP)
        nv = (off + LANES - 1) // LANES

        def bucket_of(x):
            b = jnp.minimum((x - lo) >> 11, NBUCK - 1)
            return jnp.where(x >= TAIL, NBUCK, b)

        def cntb(v, counts):
            x = match_r[pl.ds(v * LANES, LANES)]
            valid = (v * LANES + lane) < off
            b = bucket_of(x)
            return tuple(
                counts[k] + jnp.sum((valid & (b == k)).astype(jnp.int32))
                for k in range(NBUCK + 1))

        counts = lax.fori_loop(0, nv, cntb, (0,) * (NBUCK + 1))
        boff = []
        acc = 0
        for k in range(NBUCK + 1):
            boff.append(acc)
            smem[2 * k] = acc
            smem[2 * k + 1] = counts[k]
            acc = acc + counts[k] + (LANES - 1)

        def sb(v, curs):
            x = match_r[pl.ds(v * LANES, LANES)]
            p = match_p[pl.ds(v * LANES, LANES)]
            valid = (v * LANES + lane) < off
            b = bucket_of(x)
            new = []
            for k in range(NBUCK + 1):
                mk = valid & (b == k)
                ck = jnp.minimum(curs[k], M2CAP - LANES)
                plsc.store_compressed(m2r.at[pl.ds(ck, LANES)], x, mask=mk)
                plsc.store_compressed(m2p.at[pl.ds(ck, LANES)], p, mask=mk)
                new.append(curs[k] + jnp.sum(mk.astype(jnp.int32)))
            return tuple(new)

        lax.fori_loop(0, nv, sb, tuple(boff))


        def chunk_body(c, carry):
            cstart = _cstart_of(c)
            beg = smem[2 * c]
            cnt = smem[2 * c + 1]
            cnt1 = jnp.minimum(cnt, SUB)
            end = beg + cnt1
            nvx = (cnt1 + LANES - 1) // LANES
            for dg in range(8):
                p = dg % 4
                pltpu.make_async_copy(
                    tab_hbm.at[pl.ds(0, 8), pl.ds(0, BSPAN)],
                    bufs[p], sems[p]).wait()
                buf = bufs[p]

                def ext(v, carry2, _dg=dg, _buf=buf):
                    e = beg + v * LANES
                    x = m2r[pl.ds(e, LANES)]
                    valid = (e + lane) < end
                    local = jnp.where(valid, x - cstart, 0)
                    slot = v * LANES + lane
                    for dr in range(8):
                        vals = plsc.load_gather(
                            _buf,
                            [jnp.full((LANES,), dr, jnp.int32), local],
                            mask=valid)
                        plsc.store_scatter(
                            grows,
                            [slot, jnp.full((LANES,), _dg * 8 + dr,
                                            jnp.int32)],
                            vals, mask=valid)
                    return carry2

                lax.fori_loop(0, nvx, ext, 0)
                dg2 = (dg + 4) % 8
                c2 = jnp.minimum(c + (1 if dg >= 4 else 0), NBUCK - 1)
                pltpu.async_copy(
                    tab_hbm.at[pl.ds(dg2 * 8, 8), pl.ds(_cstart_of(c2), BSPAN)],
                    bufs[p], sems[p])
            def posv(v, carry2):
                e = beg + v * LANES
                pv = m2p[pl.ds(e, LANES)]
                valid = (e + lane) < end
                pos2d[0, pl.ds(v * LANES, LANES)] = jnp.where(valid, pv, -1)
                return carry2

            lax.fori_loop(0, SUB // LANES, posv, 0)
            pltpu.async_copy(
                grows,
                out_hbm.at[plsc.Indices(pos2d.at[0], ignored_value=-1)],
                sem).wait()
            return carry

        lax.fori_loop(0, NBUCK, chunk_body, 0)
        for p in range(4):
            pltpu.make_async_copy(
                tab_hbm.at[pl.ds(0, 8), pl.ds(0, BSPAN)],
                bufs[p], sems[p]).wait()

        def chunk_fb(c, carry):
            cstart = _cstart_of(c)
            beg = smem[2 * c]
            cnt = smem[2 * c + 1]
            nsb = (cnt + SUB - 1) // SUB

            def subbatch(s, carry2):
                sbeg = beg + s * SUB
                end = beg + cnt

                def fbslab(dg, carry3):
                    pltpu.sync_copy(
                        tab_hbm.at[pl.ds(pl.multiple_of(dg * 8, 8), 8),
                                   pl.ds(cstart, BSPAN)],
                        chunk_buf)

                    def ext2(v, carry4):
                        e = sbeg + v * LANES
                        x = m2r[pl.ds(e, LANES)]
                        valid = (e + lane) < end
                        local = jnp.where(valid, x - cstart, 0)
                        slot = v * LANES + lane
                        for dr in range(8):
                            vals = plsc.load_gather(
                                chunk_buf,
                                [jnp.full((LANES,), dr, jnp.int32), local],
                                mask=valid)
                            plsc.store_scatter(
                                grows,
                                [slot, dg * 8 + dr
                                 + jnp.zeros((LANES,), jnp.int32)],
                                vals, mask=valid)
                        return carry4

                    lax.fori_loop(0, SUB // LANES, ext2, 0)
                    return carry3

                lax.fori_loop(0, 8, fbslab, 0)

                def posv2(v, carry3):
                    e = sbeg + v * LANES
                    pv = m2p[pl.ds(e, LANES)]
                    valid = (e + lane) < end
                    pos2d[0, pl.ds(v * LANES, LANES)] = jnp.where(valid, pv, -1)
                    return carry3

                lax.fori_loop(0, SUB // LANES, posv2, 0)
                pltpu.async_copy(
                    grows,
                    out_hbm.at[plsc.Indices(pos2d.at[0], ignored_value=-1)],
                    sem).wait()
                return carry2

            lax.fori_loop(1, nsb, subbatch, 0)
            return carry

        lax.fori_loop(0, NBUCK, chunk_fb, 0)

        beg = smem[2 * NBUCK]
        cnt = smem[2 * NBUCK + 1]
        nsb = (cnt + SUB - 1) // SUB

        def tailbatch(s, carry2):
            sbeg = beg + s * SUB
            end = beg + cnt
            for v in range(SUB // LANES):
                e = sbeg + v * LANES
                x = m2r[pl.ds(e, LANES)]
                valid = (e + lane) < end
                local = jnp.where(valid, x - TAIL, 0)
                slot = jnp.full((LANES,), v * LANES, jnp.int32) + lane
                for d in range(DIM):
                    vals = plsc.load_gather(
                        tail_v, [local * DIM + d], mask=valid)
                    plsc.store_scatter(
                        grows, [slot, jnp.full((LANES,), d, jnp.int32)],
                        vals, mask=valid)
                p = m2p[pl.ds(e, LANES)]
                pos2d[0, pl.ds(v * LANES, LANES)] = jnp.where(valid, p, -1)
            pltpu.async_copy(
                grows,
                out_hbm.at[plsc.Indices(pos2d.at[0], ignored_value=-1)],
                sem).wait()
            return carry2

        lax.fori_loop(0, nsb, tailbatch, 0)


@functools.partial(
    pl.kernel,
    out_type=jax.ShapeDtypeStruct((BATCH,), jnp.float32),
    mesh=_mesh,
    compiler_params=pltpu.CompilerParams(needs_layout_passes=False,
                                         use_tc_tiling_on_sc=False),
    scratch_types=[
        pltpu.VMEM((BPW,), jnp.int32),
        pltpu.VMEM((BPW,), jnp.int32),
        pltpu.VMEM((BPW, DIM), jnp.float32),
        pltpu.VMEM((BPW, DIM), jnp.float32),
        pltpu.VMEM((BPW,), jnp.float32),
        pltpu.VMEM((BPW,), jnp.float32),
        pltpu.VMEM((BPW,), jnp.float32),
        pltpu.SemaphoreType.DMA,
    ],
)
def _dot(i_hbm, j_hbm, rwi_hbm, rwj_hbm, bi_hbm, bj_hbm, out_hbm,
         idx_i, idx_j, wi_rows, wj_rows, bi_rows, bj_rows, out_v, sem):
    base = _wid() * BPW
    pltpu.sync_copy(i_hbm.at[pl.ds(base, BPW)], idx_i)
    pltpu.sync_copy(j_hbm.at[pl.ds(base, BPW)], idx_j)

    copies = [
        pltpu.async_copy(rwi_hbm.at[pl.ds(base, BPW), pl.ds(0, DIM)],
                         wi_rows, sem),
        pltpu.async_copy(rwj_hbm.at[pl.ds(base, BPW), pl.ds(0, DIM)],
                         wj_rows, sem),
    ]
    for k in range(NCHUNK):
        s = pl.ds(k * CHUNK, CHUNK)
        copies.append(pltpu.async_copy(bi_hbm.at[idx_i.at[s]], bi_rows.at[s], sem))
        copies.append(pltpu.async_copy(bj_hbm.at[idx_j.at[s]], bj_rows.at[s], sem))
    for c in copies:
        c.wait()

    lane = lax.iota(jnp.int32, LANES)

    def group(g, carry):
        rows = g * LANES + lane
        acc = plsc.load_gather(bi_rows, [rows])
        acc = acc + plsc.load_gather(bj_rows, [rows])
        for d in range(DIM):
            dcol = jnp.full((LANES,), d, jnp.int32)
            acc = acc + (plsc.load_gather(wi_rows, [rows, dcol])
                         * plsc.load_gather(wj_rows, [rows, dcol]))
        out_v[pl.ds(g * LANES, LANES)] = acc
        return carry

    lax.fori_loop(0, BPW // LANES, group, 0)
    pltpu.sync_copy(out_v, out_hbm.at[pl.ds(base, BPW)])


def kernel(i_indices, j_indices, wi, wj, bi, bj):
    ii = i_indices.astype(jnp.int32)
    jj = j_indices.astype(jnp.int32)
    wi_t = wi.T
    wj_t = wj.T
    wi_tail = wi[TAIL:].reshape(-1)
    wj_tail = wj[TAIL:].reshape(-1)
    rwi, rwj = _sweep(ii, jj, wi_t, wj_t, wi_tail, wj_tail)
    return _dot(ii, jj, rwi, rwj, bi.reshape(VOCAB), bj.reshape(VOCAB))


# speedup vs baseline: 1.0134x; 1.0134x over previous
import functools

import jax
import jax.numpy as jnp
from jax import lax
from jax.experimental import pallas as pl
from jax.experimental.pallas import tpu as pltpu
from jax.experimental.pallas import tpu_sc as plsc

VOCAB = 1_000_000
DIM = 64
BATCH = 16384

NUM_CORES = 2
NUM_SUBCORES = 16
NUM_WORKERS = NUM_CORES * NUM_SUBCORES
BPW = BATCH // NUM_WORKERS
CHUNK = 128
NCHUNK = BPW // CHUNK
LANES = 16

RSPAN = 31360
BSPAN = 2048
NBUCK = 16
TAIL = 999936
MAXSTART = 997888
MCAP = 2048
M2CAP = 2432
SUB = 128
NIDX = BATCH // 8

_mesh = plsc.VectorSubcoreMesh(core_axis_name="c", subcore_axis_name="s")


def _wid():
    return lax.axis_index("s") * NUM_CORES + lax.axis_index("c")


@functools.partial(
    pl.kernel,
    out_type=(jax.ShapeDtypeStruct((BATCH, 128), jnp.float32),
              jax.ShapeDtypeStruct((BATCH, 128), jnp.float32)),
    mesh=_mesh,
    compiler_params=pltpu.CompilerParams(needs_layout_passes=False,
                                         use_tc_tiling_on_sc=True),
    scratch_types=[
        pltpu.VMEM((NIDX,), jnp.int32),
        pltpu.VMEM((MCAP + 32,), jnp.int32),
        pltpu.VMEM((MCAP + 32,), jnp.int32),
        pltpu.VMEM((M2CAP,), jnp.int32),
        pltpu.VMEM((M2CAP,), jnp.int32),
        pltpu.VMEM((8, BSPAN), jnp.float32),
        pltpu.VMEM((8, BSPAN), jnp.float32),
        pltpu.VMEM((8, BSPAN), jnp.float32),
        pltpu.VMEM((8, BSPAN), jnp.float32),
        pltpu.VMEM((SUB, 128), jnp.float32),
        pltpu.VMEM((1, 128), jnp.int32),
        pltpu.VMEM((4096,), jnp.float32),
        pltpu.SMEM((64,), jnp.int32),
        pltpu.SemaphoreType.DMA,
        pltpu.SemaphoreType.DMA,
        pltpu.SemaphoreType.DMA,
        pltpu.SemaphoreType.DMA,
        pltpu.SemaphoreType.DMA,
    ],
)
def _sweep(i_hbm, j_hbm, wi_hbm, wj_hbm, witail_hbm, wjtail_hbm,
           owi_hbm, owj_hbm,
           idx_buf, match_r, match_p, m2r, m2p, chunk_buf, chunk_buf2,
           chunk_buf3, chunk_buf4, grows, pos2d, tail_v, smem, sem,
           sem_a, sem_b, sem_c, sem_d):
    wid = _wid()
    lo = wid * RSPAN
    hi = jnp.minimum(lo + RSPAN, VOCAB)
    lane = lax.iota(jnp.int32, LANES)

    for t in range(2):
        idx_hbm = (i_hbm, j_hbm)[t]
        tab_hbm = (wi_hbm, wj_hbm)[t]
        tl_hbm = (witail_hbm, wjtail_hbm)[t]
        out_hbm = (owi_hbm, owj_hbm)[t]

        pltpu.sync_copy(tl_hbm, tail_v)

        bufs = (chunk_buf, chunk_buf2, chunk_buf3, chunk_buf4)
        sems = (sem_a, sem_b, sem_c, sem_d)

        def _cstart_of(c):
            return pl.multiple_of(jnp.minimum(lo + c * BSPAN, MAXSTART), 128)

        cs0 = _cstart_of(jnp.int32(0))
        for p in range(4):
            pltpu.async_copy(tab_hbm.at[pl.ds(p * 8, 8), pl.ds(cs0, BSPAN)],
                             bufs[p], sems[p])

        def outer(c8, off):
            pltpu.sync_copy(idx_hbm.at[pl.ds(c8 * NIDX, NIDX)], idx_buf)

            def cb(v, off):
                x = idx_buf[pl.ds(v * LANES, LANES)]
                m = (x >= lo) & (x < hi)
                cnt = jnp.sum(m.astype(jnp.int32))
                offw = jnp.minimum(off, MCAP)
                plsc.store_compressed(match_r.at[pl.ds(offw, LANES)], x, mask=m)
                pos = c8 * NIDX + v * LANES + lane
                plsc.store_compressed(match_p.at[pl.ds(offw, LANES)], pos, mask=m)
                return off + cnt

            return lax.fori_loop(0, NIDX // LANES, cb, off)

        off = lax.fori_loop(0, 8, outer, 0)
        off = jnp.minimum(off, MCAP)
        nv = (off + LANES - 1) // LANES

        def bucket_of(x):
            b = jnp.minimum((x - lo) >> 11, NBUCK - 1)
            return jnp.where(x >= TAIL, NBUCK, b)

        def cntb(v, counts):
            x = match_r[pl.ds(v * LANES, LANES)]
            valid = (v * LANES + lane) < off
            b = bucket_of(x)
            return tuple(
                counts[k] + jnp.sum((valid & (b == k)).astype(jnp.int32))
                for k in range(NBUCK + 1))

        counts = lax.fori_loop(0, nv, cntb, (0,) * (NBUCK + 1))
        boff = []
        acc = 0
        for k in range(NBUCK + 1):
            boff.append(acc)
            smem[2 * k] = acc
            smem[2 * k + 1] = counts[k]
            acc = acc + counts[k] + (LANES - 1)

        def sb(v, curs):
            x = match_r[pl.ds(v * LANES, LANES)]
            p = match_p[pl.ds(v * LANES, LANES)]
            valid = (v * LANES + lane) < off
            b = bucket_of(x)
            new = []
            for k in range(NBUCK + 1):
                mk = valid & (b == k)
                ck = jnp.minimum(curs[k], M2CAP - LANES)
                plsc.store_compressed(m2r.at[pl.ds(ck, LANES)], x, mask=mk)
                plsc.store_compressed(m2p.at[pl.ds(ck, LANES)], p, mask=mk)
                new.append(curs[k] + jnp.sum(mk.astype(jnp.int32)))
            return tuple(new)

        lax.fori_loop(0, nv, sb, tuple(boff))


        def chunk_body(c, carry):
            cstart = _cstart_of(c)
            beg = smem[2 * c]
            cnt = smem[2 * c + 1]
            end = beg + jnp.minimum(cnt, SUB)
            for dg in range(8):
                p = dg % 4
                pltpu.make_async_copy(
                    tab_hbm.at[pl.ds(0, 8), pl.ds(0, BSPAN)],
                    bufs[p], sems[p]).wait()
                buf = bufs[p]

                def ext(v, carry2, _dg=dg, _buf=buf):
                    e = beg + v * LANES
                    x = m2r[pl.ds(e, LANES)]
                    valid = (e + lane) < end
                    local = jnp.where(valid, x - cstart, 0)
                    slot = v * LANES + lane
                    for dr in range(8):
                        vals = plsc.load_gather(
                            _buf,
                            [jnp.full((LANES,), dr, jnp.int32), local],
                            mask=valid)
                        plsc.store_scatter(
                            grows,
                            [slot, jnp.full((LANES,), _dg * 8 + dr,
                                            jnp.int32)],
                            vals, mask=valid)
                    return carry2

                lax.fori_loop(0, SUB // LANES, ext, 0)
                dg2 = (dg + 4) % 8
                c2 = jnp.minimum(c + (1 if dg >= 4 else 0), NBUCK - 1)
                pltpu.async_copy(
                    tab_hbm.at[pl.ds(dg2 * 8, 8), pl.ds(_cstart_of(c2), BSPAN)],
                    bufs[p], sems[p])
            def posv(v, carry2):
                e = beg + v * LANES
                pv = m2p[pl.ds(e, LANES)]
                valid = (e + lane) < end
                pos2d[0, pl.ds(v * LANES, LANES)] = jnp.where(valid, pv, -1)
                return carry2

            lax.fori_loop(0, SUB // LANES, posv, 0)
            pltpu.async_copy(
                grows,
                out_hbm.at[plsc.Indices(pos2d.at[0], ignored_value=-1)],
                sem).wait()
            return carry

        lax.fori_loop(0, NBUCK, chunk_body, 0)
        for p in range(4):
            pltpu.make_async_copy(
                tab_hbm.at[pl.ds(0, 8), pl.ds(0, BSPAN)],
                bufs[p], sems[p]).wait()

        def chunk_fb(c, carry):
            cstart = _cstart_of(c)
            beg = smem[2 * c]
            cnt = smem[2 * c + 1]
            nsb = (cnt + SUB - 1) // SUB

            def subbatch(s, carry2):
                sbeg = beg + s * SUB
                end = beg + cnt

                def fbslab(dg, carry3):
                    pltpu.sync_copy(
                        tab_hbm.at[pl.ds(pl.multiple_of(dg * 8, 8), 8),
                                   pl.ds(cstart, BSPAN)],
                        chunk_buf)

                    def ext2(v, carry4):
                        e = sbeg + v * LANES
                        x = m2r[pl.ds(e, LANES)]
                        valid = (e + lane) < end
                        local = jnp.where(valid, x - cstart, 0)
                        slot = v * LANES + lane
                        for dr in range(8):
                            vals = plsc.load_gather(
                                chunk_buf,
                                [jnp.full((LANES,), dr, jnp.int32), local],
                                mask=valid)
                            plsc.store_scatter(
                                grows,
                                [slot, dg * 8 + dr
                                 + jnp.zeros((LANES,), jnp.int32)],
                                vals, mask=valid)
                        return carry4

                    lax.fori_loop(0, SUB // LANES, ext2, 0)
                    return carry3

                lax.fori_loop(0, 8, fbslab, 0)

                def posv2(v, carry3):
                    e = sbeg + v * LANES
                    pv = m2p[pl.ds(e, LANES)]
                    valid = (e + lane) < end
                    pos2d[0, pl.ds(v * LANES, LANES)] = jnp.where(valid, pv, -1)
                    return carry3

                lax.fori_loop(0, SUB // LANES, posv2, 0)
                pltpu.async_copy(
                    grows,
                    out_hbm.at[plsc.Indices(pos2d.at[0], ignored_value=-1)],
                    sem).wait()
                return carry2

            lax.fori_loop(1, nsb, subbatch, 0)
            return carry

        lax.fori_loop(0, NBUCK, chunk_fb, 0)

        beg = smem[2 * NBUCK]
        cnt = smem[2 * NBUCK + 1]
        nsb = (cnt + SUB - 1) // SUB

        def tailbatch(s, carry2):
            sbeg = beg + s * SUB
            end = beg + cnt
            for v in range(SUB // LANES):
                e = sbeg + v * LANES
                x = m2r[pl.ds(e, LANES)]
                valid = (e + lane) < end
                local = jnp.where(valid, x - TAIL, 0)
                slot = jnp.full((LANES,), v * LANES, jnp.int32) + lane
                for d in range(DIM):
                    vals = plsc.load_gather(
                        tail_v, [local * DIM + d], mask=valid)
                    plsc.store_scatter(
                        grows, [slot, jnp.full((LANES,), d, jnp.int32)],
                        vals, mask=valid)
                p = m2p[pl.ds(e, LANES)]
                pos2d[0, pl.ds(v * LANES, LANES)] = jnp.where(valid, p, -1)
            pltpu.async_copy(
                grows,
                out_hbm.at[plsc.Indices(pos2d.at[0], ignored_value=-1)],
                sem).wait()
            return carry2

        lax.fori_loop(0, nsb, tailbatch, 0)


@functools.partial(
    pl.kernel,
    out_type=jax.ShapeDtypeStruct((BATCH,), jnp.float32),
    mesh=_mesh,
    compiler_params=pltpu.CompilerParams(needs_layout_passes=False,
                                         use_tc_tiling_on_sc=False),
    scratch_types=[
        pltpu.VMEM((BPW,), jnp.int32),
        pltpu.VMEM((BPW,), jnp.int32),
        pltpu.VMEM((BPW, DIM), jnp.float32),
        pltpu.VMEM((BPW, DIM), jnp.float32),
        pltpu.VMEM((BPW,), jnp.float32),
        pltpu.VMEM((BPW,), jnp.float32),
        pltpu.VMEM((BPW,), jnp.float32),
        pltpu.SemaphoreType.DMA,
    ],
)
def _dot(i_hbm, j_hbm, rwi_hbm, rwj_hbm, bi_hbm, bj_hbm, out_hbm,
         idx_i, idx_j, wi_rows, wj_rows, bi_rows, bj_rows, out_v, sem):
    base = _wid() * BPW
    pltpu.sync_copy(i_hbm.at[pl.ds(base, BPW)], idx_i)
    pltpu.sync_copy(j_hbm.at[pl.ds(base, BPW)], idx_j)

    copies = [
        pltpu.async_copy(rwi_hbm.at[pl.ds(base, BPW), pl.ds(0, DIM)],
                         wi_rows, sem),
        pltpu.async_copy(rwj_hbm.at[pl.ds(base, BPW), pl.ds(0, DIM)],
                         wj_rows, sem),
    ]
    for k in range(NCHUNK):
        s = pl.ds(k * CHUNK, CHUNK)
        copies.append(pltpu.async_copy(bi_hbm.at[idx_i.at[s]], bi_rows.at[s], sem))
        copies.append(pltpu.async_copy(bj_hbm.at[idx_j.at[s]], bj_rows.at[s], sem))
    for c in copies:
        c.wait()

    lane = lax.iota(jnp.int32, LANES)

    def group(g, carry):
        rows = g * LANES + lane
        acc = plsc.load_gather(bi_rows, [rows])
        acc = acc + plsc.load_gather(bj_rows, [rows])
        for d in range(DIM):
            dcol = jnp.full((LANES,), d, jnp.int32)
            acc = acc + (plsc.load_gather(wi_rows, [rows, dcol])
                         * plsc.load_gather(wj_rows, [rows, dcol]))
        out_v[pl.ds(g * LANES, LANES)] = acc
        return carry

    lax.fori_loop(0, BPW // LANES, group, 0)
    pltpu.sync_copy(out_v, out_hbm.at[pl.ds(base, BPW)])


def kernel(i_indices, j_indices, wi, wj, bi, bj):
    ii = i_indices.astype(jnp.int32)
    jj = j_indices.astype(jnp.int32)
    wi_t = wi.T
    wj_t = wj.T
    wi_tail = wi[TAIL:].reshape(-1)
    wj_tail = wj[TAIL:].reshape(-1)
    rwi, rwj = _sweep(ii, jj, wi_t, wj_t, wi_tail, wj_tail)
    return _dot(ii, jj, rwi, rwj, bi.reshape(VOCAB), bj.reshape(VOCAB))
